# baseline (speedup 1.0000x reference)
import functools

import jax
import jax.numpy as jnp
from jax import lax
from jax.experimental import pallas as pl
from jax.experimental.pallas import tpu as pltpu
from jax.experimental.pallas import tpu_sc as plsc

NC = 2
NS = 16
BATCH = 16384
DIM = 64
ROWS = 100000
CPT = DIM // NS
L = 16
QUARTER = BATCH // 4


def _body(users_hbm, items_hbm, utab_hbm, itab_hbm, uout_hbm, iout_hbm,
          idx_v, slab_v, out_v):
    sc = lax.axis_index("c")
    sid = lax.axis_index("s")

    def run(idx_hbm, tab_hbm, out_hbm):
        pltpu.sync_copy(idx_hbm, idx_v)
        for j in range(CPT):
            col = sid * CPT + j
            pltpu.sync_copy(tab_hbm.at[col, pl.ds(0, ROWS)], slab_v)

            def quarter(q, _):
                def step(k, _):
                    off = q * QUARTER + k * L
                    idx16 = idx_v[pl.ds(off, L)]
                    out_v[pl.ds(k * L, L)] = plsc.load_gather(slab_v, [idx16])
                    return 0
                lax.fori_loop(0, QUARTER // L, step, 0, unroll=8)
                pltpu.sync_copy(
                    out_v, out_hbm.at[col, pl.ds(q * QUARTER, QUARTER)])
                return 0

            lax.fori_loop(0, 4, quarter, 0)

    @pl.when(sc == 0)
    def _():
        run(users_hbm, utab_hbm, uout_hbm)

    @pl.when(sc == 1)
    def _():
        run(items_hbm, itab_hbm, iout_hbm)


@functools.cache
def _build():
    mesh = plsc.VectorSubcoreMesh(core_axis_name="c", subcore_axis_name="s",
                                  num_cores=NC, num_subcores=NS)
    return pl.kernel(
        _body,
        out_type=(jax.ShapeDtypeStruct((DIM, BATCH), jnp.float32),
                  jax.ShapeDtypeStruct((DIM, BATCH), jnp.float32)),
        mesh=mesh,
        scratch_types=[
            pltpu.VMEM((BATCH,), jnp.int32),
            pltpu.VMEM((ROWS,), jnp.float32),
            pltpu.VMEM((QUARTER,), jnp.float32),
        ],
        compiler_params=pltpu.CompilerParams(use_tc_tiling_on_sc=False,
                                             needs_layout_passes=False),
    )


@jax.jit
def kernel(users, items, user_table, item_table):
    uout_t, iout_t = _build()(jnp.asarray(users, jnp.int32),
                              jnp.asarray(items, jnp.int32),
                              user_table.T, item_table.T)
    return uout_t.T, iout_t.T


# speedup vs baseline: 1.0145x; 1.0145x over previous
import functools

import jax
import jax.numpy as jnp
from jax import lax
from jax.experimental import pallas as pl
from jax.experimental.pallas import tpu as pltpu
from jax.experimental.pallas import tpu_sc as plsc

NC = 2
NS = 16
BATCH = 16384
DIM = 64
ROWS = 100000
CPT = DIM // NS
CH = 128
INFLIGHT = 16
NGRP = BATCH // (CH * INFLIGHT)
HALF = BATCH // 2


def _body(users_hbm, items_hbm, utab_hbm, itab_hbm, uout_hbm, iout_hbm,
          idx_v, out_v, slab_sh, gsem):
    sc = lax.axis_index("c")
    sid = lax.axis_index("s")
    slab = slab_sh.at[pl.ds(sid * ROWS, ROWS)]

    def run(idx_hbm, tab_hbm, out_hbm):
        pltpu.sync_copy(idx_hbm, idx_v)
        for j in range(CPT):
            col = sid * CPT + j
            pltpu.sync_copy(tab_hbm.at[col, pl.ds(0, ROWS)], slab)

            for h in range(2):
                def grp(g, _, h=h):
                    handles = []
                    for t in range(INFLIGHT):
                        off = (g * INFLIGHT + t) * CH
                        handles.append(pltpu.async_copy(
                            slab.at[idx_v.at[pl.ds(h * HALF + off, CH)]],
                            out_v.at[pl.ds(off, CH)], gsem))
                    for hh in handles:
                        hh.wait()
                    return 0

                lax.fori_loop(0, NGRP // 2, grp, 0)
                pltpu.sync_copy(
                    out_v, out_hbm.at[col, pl.ds(h * HALF, HALF)])

    @pl.when(sc == 0)
    def _():
        run(users_hbm, utab_hbm, uout_hbm)

    @pl.when(sc == 1)
    def _():
        run(items_hbm, itab_hbm, iout_hbm)


@functools.cache
def _build():
    mesh = plsc.VectorSubcoreMesh(core_axis_name="c", subcore_axis_name="s",
                                  num_cores=NC, num_subcores=NS)
    return pl.kernel(
        _body,
        out_type=(jax.ShapeDtypeStruct((DIM, BATCH), jnp.float32),
                  jax.ShapeDtypeStruct((DIM, BATCH), jnp.float32)),
        mesh=mesh,
        scratch_types=[
            pltpu.VMEM((BATCH,), jnp.int32),
            pltpu.VMEM((HALF,), jnp.float32),
            pltpu.MemorySpace.VMEM_SHARED((NS * ROWS,), jnp.float32),
            pltpu.SemaphoreType.DMA,
        ],
        compiler_params=pltpu.CompilerParams(use_tc_tiling_on_sc=False,
                                             needs_layout_passes=False),
    )


@jax.jit
def kernel(users, items, user_table, item_table):
    uout_t, iout_t = _build()(jnp.asarray(users, jnp.int32),
                              jnp.asarray(items, jnp.int32),
                              user_table.T, item_table.T)
    return uout_t.T, iout_t.T


# speedup vs baseline: 4.9170x; 4.8467x over previous
import functools

import jax
import jax.numpy as jnp
from jax import lax
from jax.experimental import pallas as pl
from jax.experimental.pallas import tpu as pltpu
from jax.experimental.pallas import tpu_sc as plsc

NC = 2
NS = 16
NW = NC * NS
BATCH = 16384
DIM = 64
PDIM = 128
BPW = BATCH // NW
CH = 128
NCH = BPW // CH
NBUF = 4


def _body(idx_hbm, tab_hbm, out_hbm, idx_v, b0, b1, b2, b3,
          gs0, gs1, gs2, gs3, os0, os1, os2, os3):
    bufs = [b0, b1, b2, b3]
    gsems = [gs0, gs1, gs2, gs3]
    osems = [os0, os1, os2, os3]
    wid = lax.axis_index("s") * NC + lax.axis_index("c")
    base = wid * BPW
    pltpu.sync_copy(idx_hbm.at[pl.ds(base, BPW)], idx_v)

    def gather(c, b):
        return pltpu.async_copy(
            tab_hbm.at[idx_v.at[pl.ds(c * CH, CH)]], bufs[b], gsems[b])

    def out(c):
        b = c % NBUF
        return pltpu.async_copy(
            bufs[b], out_hbm.at[pl.ds(base + c * CH, CH)], osems[b])

    g_h = [None] * NCH
    o_h = [None] * NCH
    for c in range(NCH):
        if c >= NBUF:
            o_h[c - NBUF].wait()
        g_h[c] = gather(c, c % NBUF)
        if c >= 2:
            g_h[c - 2].wait()
            o_h[c - 2] = out(c - 2)
    for c in range(NCH - 2, NCH):
        g_h[c].wait()
        o_h[c] = out(c)
    for c in range(max(0, NCH - NBUF), NCH):
        o_h[c].wait()


@functools.cache
def _build():
    mesh = plsc.VectorSubcoreMesh(core_axis_name="c", subcore_axis_name="s",
                                  num_cores=NC, num_subcores=NS)
    return pl.kernel(
        _body,
        out_type=jax.ShapeDtypeStruct((BATCH, PDIM), jnp.float32),
        mesh=mesh,
        scratch_types=[
            pltpu.VMEM((BPW,), jnp.int32),
            *[pltpu.VMEM((CH, PDIM), jnp.float32) for _ in range(NBUF)],
            *[pltpu.SemaphoreType.DMA for _ in range(2 * NBUF)],
        ],
    )


@jax.jit
def kernel(users, items, user_table, item_table):
    gath = _build()
    utab = jnp.pad(user_table, ((0, 0), (0, PDIM - DIM)))
    uout = gath(jnp.asarray(users, jnp.int32), utab)
    itab = jnp.pad(item_table, ((0, 0), (0, PDIM - DIM)))
    iout = gath(jnp.asarray(items, jnp.int32), itab)
    return uout[:, :DIM], iout[:, :DIM]
